# baseline (speedup 1.0000x reference)
import functools

import jax
import jax.numpy as jnp
from jax import lax
from jax.experimental import pallas as pl
from jax.experimental.pallas import tpu as pltpu
from jax.experimental.pallas import tpu_sc as plsc

EPS_LN = 1e-6

_SC_CORES = 2
_SC_SUBCORES = 16
_SC_LANES = 16
_NW = _SC_CORES * _SC_SUBCORES


def _knn_body(npow2_bits, ct_ref, xt_ref, i_ref, w_ref):
    n = ct_ref.shape[2]
    tm = xt_ref.shape[1]
    cx = ct_ref[0, 0:1, :]
    cy = ct_ref[0, 1:2, :]
    cz = ct_ref[0, 2:3, :]
    tx = xt_ref[0, :, 0:1]
    ty = xt_ref[0, :, 1:2]
    tz = xt_ref[0, :, 2:3]
    dx = tx - cx
    dy = ty - cy
    dz = tz - cz
    d2 = dx * dx + dy * dy + dz * dz
    nb = 128
    bigf = jnp.float32(3e9)
    m1 = jnp.full((tm, nb), bigf, jnp.float32)
    m2 = m1
    m3 = m1
    flag = 1 << 30
    for g in range(n // nb):
        lane_c = (jax.lax.broadcasted_iota(jnp.int32, (tm, nb), 1)
                  + (g * nb | flag))
        kc_i = (d2[:, g * nb:(g + 1) * nb] << npow2_bits) | lane_c
        kc = jax.lax.bitcast_convert_type(kc_i, jnp.float32)
        lo = jnp.minimum(m1, kc)
        hi = jnp.maximum(m1, kc)
        m1 = lo
        lo = jnp.minimum(m2, hi)
        hi = jnp.maximum(m2, hi)
        m2 = lo
        m3 = jnp.minimum(m3, hi)
    ks = jnp.concatenate([m1, m2, m3], axis=1)
    k1 = jnp.min(ks, axis=1, keepdims=True)
    ks = jnp.where(ks == k1, bigf, ks)
    k2 = jnp.min(ks, axis=1, keepdims=True)
    ks = jnp.where(ks == k2, bigf, ks)
    k3 = jnp.min(ks, axis=1, keepdims=True)

    mask = jnp.int32((1 << npow2_bits) - 1)
    w_list = []
    idx_list = []
    for k in (k1, k2, k3):
        ki = jax.lax.bitcast_convert_type(k, jnp.int32) & (flag - 1)
        idx_list.append(ki & mask)
        d2f = (ki >> npow2_bits).astype(jnp.float32)
        w_list.append(1.0 / (d2f + 1e-8))
    norm = w_list[0] + w_list[1] + w_list[2]
    w_list = [w / norm for w in w_list]

    zi = jnp.zeros((tm, 5), jnp.int32)
    zf = jnp.zeros((tm, 5), jnp.float32)
    i_ref[0] = jnp.concatenate(idx_list + [zi], axis=1)
    w_ref[0] = jnp.concatenate(w_list + [zf], axis=1)


def _combine_body(g_ref, w_ref, ga_ref, be_ref, o_ref):
    acc = (g_ref[0] * w_ref[:, 0:1]
           + g_ref[1] * w_ref[:, 1:2]
           + g_ref[2] * w_ref[:, 2:3])
    mu = jnp.mean(acc, axis=1, keepdims=True)
    xc = acc - mu
    var = jnp.mean(xc * xc, axis=1, keepdims=True)
    y = xc / jnp.sqrt(var + EPS_LN) * ga_ref[0:1, :] + be_ref[0:1, :]
    o_ref[...] = y * 0.5 * (1.0 + jax.lax.erf(y * 0.7071067811865476))


def _make_sc_gather(b, n, m, c, r):
    rw = r // _NW
    g = 128
    wpb = m // rw
    mesh = plsc.VectorSubcoreMesh(core_axis_name="c", subcore_axis_name="s")

    @functools.partial(
        pl.kernel,
        mesh=mesh,
        compiler_params=pltpu.CompilerParams(needs_layout_passes=False),
        out_type=jax.ShapeDtypeStruct((3, r, c), jnp.float32),
        scratch_types=[
            pltpu.VMEM((n,), jnp.int32),
            pltpu.VMEM((rw,), jnp.int32),
            pltpu.VMEM((g,), jnp.int32),
            pltpu.VMEM((g, c), jnp.float32),
            pltpu.SemaphoreType.DMA,
        ],
    )
    def sc_gather(order_hbm, s0_hbm, s1_hbm, s2_hbm, feats_hbm, out_hbm,
                  order_v, sidx_v, gidx_v, rows_v, sem):
        wid = lax.axis_index("s") * _SC_CORES + lax.axis_index("c")
        base = wid * rw
        batch = wid // wpb
        bn = batch * n
        pltpu.sync_copy(order_hbm.at[pl.ds(bn, n)], order_v)
        for j, s_hbm in enumerate((s0_hbm, s1_hbm, s2_hbm)):
            pltpu.sync_copy(s_hbm.at[pl.ds(base, rw)], sidx_v)
            for ch in range(rw // g):
                for q in range(g // _SC_LANES):
                    sl = pl.ds(q * _SC_LANES, _SC_LANES)
                    iv = sidx_v[pl.ds(ch * g + q * _SC_LANES, _SC_LANES)]
                    gidx_v[sl] = plsc.load_gather(order_v, [iv]) + bn
                pltpu.async_copy(feats_hbm.at[gidx_v], rows_v, sem).wait()
                pltpu.sync_copy(rows_v, out_hbm.at[j, pl.ds(base + ch * g, g)])

    return sc_gather


def _sort_keys(coord):
    step = coord.max() + 1
    return coord[..., 0] + coord[..., 1] * step + coord[..., 2] * step * step


def kernel(features, coords, xyz_t, gamma, beta):
    b, n, c = features.shape
    m = xyz_t.shape[1]
    r = b * m
    npow2_bits = max(1, (n - 1).bit_length())

    order = jnp.argsort(_sort_keys(coords), axis=1)
    order_t = jnp.argsort(_sort_keys(xyz_t), axis=1)
    c_sorted = jnp.take_along_axis(coords, order[..., None], axis=1)
    xt_s = jnp.take_along_axis(xyz_t, order_t[..., None], axis=1)

    ct_pad = jnp.zeros((b, 8, n), jnp.int32).at[:, 0:3, :].set(
        c_sorted.transpose(0, 2, 1))
    xt_pad = jnp.zeros((b, m, 8), jnp.int32).at[:, :, 0:3].set(xt_s)

    tm = 256
    sidx8, w8 = pl.pallas_call(
        functools.partial(_knn_body, npow2_bits),
        grid=(b, m // tm),
        in_specs=[
            pl.BlockSpec((1, 8, n), lambda i, j: (i, 0, 0)),
            pl.BlockSpec((1, tm, 8), lambda i, j: (i, j, 0)),
        ],
        out_specs=[
            pl.BlockSpec((1, tm, 8), lambda i, j: (i, j, 0)),
            pl.BlockSpec((1, tm, 8), lambda i, j: (i, j, 0)),
        ],
        out_shape=[
            jax.ShapeDtypeStruct((b, m, 8), jnp.int32),
            jax.ShapeDtypeStruct((b, m, 8), jnp.float32),
        ],
    )(ct_pad, xt_pad)

    sidx_flat = sidx8[:, :, 0:3].reshape(r, 3)
    sidx3 = sidx8[:, :, 0:3]
    oidx = jnp.take_along_axis(order, sidx3.reshape(b, m * 3), axis=1)
    gathered = jnp.take_along_axis(
        features, oidx[..., None], axis=1).reshape(b, m, 3, c)
    gathered = gathered.transpose(2, 0, 1, 3).reshape(3, r, c)

    tr = 1024
    out_feats = pl.pallas_call(
        _combine_body,
        grid=(r // tr,),
        in_specs=[
            pl.BlockSpec((3, tr, c), lambda i: (0, i, 0)),
            pl.BlockSpec((tr, 8), lambda i: (i, 0)),
            pl.BlockSpec((1, c), lambda i: (0, 0)),
            pl.BlockSpec((1, c), lambda i: (0, 0)),
        ],
        out_specs=pl.BlockSpec((tr, c), lambda i: (i, 0)),
        out_shape=jax.ShapeDtypeStruct((r, c), jnp.float32),
    )(gathered, w8.reshape(r, 8), gamma.reshape(1, c), beta.reshape(1, c))

    xt_f = xt_s.reshape(r, 3).astype(jnp.float32)
    bcol = jnp.repeat(jnp.arange(b, dtype=jnp.float32), m)[:, None]
    out_coords = jnp.concatenate([bcol, xt_f], axis=1)
    return out_feats, out_coords


# speedup vs baseline: 5.5220x; 5.5220x over previous
import functools

import jax
import jax.numpy as jnp
from jax import lax
from jax.experimental import pallas as pl
from jax.experimental.pallas import tpu as pltpu
from jax.experimental.pallas import tpu_sc as plsc

EPS_LN = 1e-6

_SC_CORES = 2
_SC_SUBCORES = 16
_SC_LANES = 16
_NW = _SC_CORES * _SC_SUBCORES


def _knn_body(npow2_bits, ct_ref, xt_ref, i_ref, w_ref):
    n = ct_ref.shape[2]
    tm = xt_ref.shape[1]
    cx = ct_ref[0, 0:1, :]
    cy = ct_ref[0, 1:2, :]
    cz = ct_ref[0, 2:3, :]
    tx = xt_ref[0, :, 0:1]
    ty = xt_ref[0, :, 1:2]
    tz = xt_ref[0, :, 2:3]
    dx = tx - cx
    dy = ty - cy
    dz = tz - cz
    d2 = dx * dx + dy * dy + dz * dz
    nb = 128
    bigf = jnp.float32(3e9)
    m1 = jnp.full((tm, nb), bigf, jnp.float32)
    m2 = m1
    m3 = m1
    flag = 1 << 30
    for g in range(n // nb):
        lane_c = (jax.lax.broadcasted_iota(jnp.int32, (tm, nb), 1)
                  + (g * nb | flag))
        kc_i = (d2[:, g * nb:(g + 1) * nb] << npow2_bits) | lane_c
        kc = jax.lax.bitcast_convert_type(kc_i, jnp.float32)
        lo = jnp.minimum(m1, kc)
        hi = jnp.maximum(m1, kc)
        m1 = lo
        lo = jnp.minimum(m2, hi)
        hi = jnp.maximum(m2, hi)
        m2 = lo
        m3 = jnp.minimum(m3, hi)
    ks = jnp.concatenate([m1, m2, m3], axis=1)
    k1 = jnp.min(ks, axis=1, keepdims=True)
    ks = jnp.where(ks == k1, bigf, ks)
    k2 = jnp.min(ks, axis=1, keepdims=True)
    ks = jnp.where(ks == k2, bigf, ks)
    k3 = jnp.min(ks, axis=1, keepdims=True)

    mask = jnp.int32((1 << npow2_bits) - 1)
    w_list = []
    idx_list = []
    for k in (k1, k2, k3):
        ki = jax.lax.bitcast_convert_type(k, jnp.int32) & (flag - 1)
        idx_list.append(ki & mask)
        d2f = (ki >> npow2_bits).astype(jnp.float32)
        w_list.append(1.0 / (d2f + 1e-8))
    norm = w_list[0] + w_list[1] + w_list[2]
    w_list = [w / norm for w in w_list]

    zi = jnp.zeros((tm, 5), jnp.int32)
    zf = jnp.zeros((tm, 5), jnp.float32)
    i_ref[0] = jnp.concatenate(idx_list + [zi], axis=1)
    w_ref[0] = jnp.concatenate(w_list + [zf], axis=1)


def _combine_body(g0_ref, g1_ref, g2_ref, w_ref, ga_ref, be_ref, o_ref):
    acc = (g0_ref[...] * w_ref[:, 0:1]
           + g1_ref[...] * w_ref[:, 1:2]
           + g2_ref[...] * w_ref[:, 2:3])
    mu = jnp.mean(acc, axis=1, keepdims=True)
    xc = acc - mu
    var = jnp.mean(xc * xc, axis=1, keepdims=True)
    y = xc / jnp.sqrt(var + EPS_LN) * ga_ref[0:1, :] + be_ref[0:1, :]
    o_ref[...] = y * 0.5 * (1.0 + jax.lax.erf(y * 0.7071067811865476))


def _make_sc_gather(b, n, m, c, r):
    rw = r // _NW
    g = 128
    wpb = m // rw
    mesh = plsc.VectorSubcoreMesh(core_axis_name="c", subcore_axis_name="s")

    out_row = jax.ShapeDtypeStruct((r, c), jnp.float32)

    @functools.partial(
        pl.kernel,
        mesh=mesh,
        compiler_params=pltpu.CompilerParams(needs_layout_passes=False),
        out_type=(out_row, out_row, out_row),
        scratch_types=[
            pltpu.VMEM((n,), jnp.int32),
            pltpu.VMEM((rw,), jnp.int32),
            pltpu.VMEM((g,), jnp.int32),
            pltpu.VMEM((g, c), jnp.float32),
            pltpu.SemaphoreType.DMA,
        ],
    )
    def sc_gather(order_hbm, s0_hbm, s1_hbm, s2_hbm, feats_hbm,
                  o0_hbm, o1_hbm, o2_hbm,
                  order_v, sidx_v, gidx_v, rows_v, sem):
        wid = lax.axis_index("s") * _SC_CORES + lax.axis_index("c")
        base = wid * rw
        batch = wid // wpb
        bn = batch * n
        pltpu.sync_copy(order_hbm.at[pl.ds(bn, n)], order_v)
        for s_hbm, o_hbm in ((s0_hbm, o0_hbm), (s1_hbm, o1_hbm),
                             (s2_hbm, o2_hbm)):
            pltpu.sync_copy(s_hbm.at[pl.ds(base, rw)], sidx_v)
            for ch in range(rw // g):
                for q in range(g // _SC_LANES):
                    sl = pl.ds(q * _SC_LANES, _SC_LANES)
                    iv = sidx_v[pl.ds(ch * g + q * _SC_LANES, _SC_LANES)]
                    gidx_v[sl] = plsc.load_gather(order_v, [iv]) + bn
                pltpu.async_copy(feats_hbm.at[gidx_v], rows_v, sem).wait()
                pltpu.sync_copy(rows_v, o_hbm.at[pl.ds(base + ch * g, g)])

    return sc_gather


def _sort_keys(coord):
    step = coord.max() + 1
    return coord[..., 0] + coord[..., 1] * step + coord[..., 2] * step * step


def kernel(features, coords, xyz_t, gamma, beta):
    b, n, c = features.shape
    m = xyz_t.shape[1]
    r = b * m
    npow2_bits = max(1, (n - 1).bit_length())

    order = jnp.argsort(_sort_keys(coords), axis=1)
    order_t = jnp.argsort(_sort_keys(xyz_t), axis=1)
    c_sorted = jnp.take_along_axis(coords, order[..., None], axis=1)
    xt_s = jnp.take_along_axis(xyz_t, order_t[..., None], axis=1)

    ct_pad = jnp.zeros((b, 8, n), jnp.int32).at[:, 0:3, :].set(
        c_sorted.transpose(0, 2, 1))
    xt_pad = jnp.zeros((b, m, 8), jnp.int32).at[:, :, 0:3].set(xt_s)

    tm = 256
    sidx8, w8 = pl.pallas_call(
        functools.partial(_knn_body, npow2_bits),
        grid=(b, m // tm),
        in_specs=[
            pl.BlockSpec((1, 8, n), lambda i, j: (i, 0, 0)),
            pl.BlockSpec((1, tm, 8), lambda i, j: (i, j, 0)),
        ],
        out_specs=[
            pl.BlockSpec((1, tm, 8), lambda i, j: (i, j, 0)),
            pl.BlockSpec((1, tm, 8), lambda i, j: (i, j, 0)),
        ],
        out_shape=[
            jax.ShapeDtypeStruct((b, m, 8), jnp.int32),
            jax.ShapeDtypeStruct((b, m, 8), jnp.float32),
        ],
    )(ct_pad, xt_pad)

    sidx_flat = sidx8[:, :, 0:3].reshape(r, 3)
    g0, g1, g2 = _make_sc_gather(b, n, m, c, r)(
        order.astype(jnp.int32).reshape(b * n),
        sidx_flat[:, 0], sidx_flat[:, 1], sidx_flat[:, 2],
        features.reshape(b * n, c))

    tr = 1024
    out_feats = pl.pallas_call(
        _combine_body,
        grid=(r // tr,),
        in_specs=[
            pl.BlockSpec((tr, c), lambda i: (i, 0)),
            pl.BlockSpec((tr, c), lambda i: (i, 0)),
            pl.BlockSpec((tr, c), lambda i: (i, 0)),
            pl.BlockSpec((tr, 8), lambda i: (i, 0)),
            pl.BlockSpec((1, c), lambda i: (0, 0)),
            pl.BlockSpec((1, c), lambda i: (0, 0)),
        ],
        out_specs=pl.BlockSpec((tr, c), lambda i: (i, 0)),
        out_shape=jax.ShapeDtypeStruct((r, c), jnp.float32),
    )(g0, g1, g2, w8.reshape(r, 8), gamma.reshape(1, c), beta.reshape(1, c))

    xt_f = xt_s.reshape(r, 3).astype(jnp.float32)
    bcol = jnp.repeat(jnp.arange(b, dtype=jnp.float32), m)[:, None]
    out_coords = jnp.concatenate([bcol, xt_f], axis=1)
    return out_feats, out_coords


# speedup vs baseline: 5.7773x; 1.0462x over previous
import functools

import jax
import jax.numpy as jnp
from jax import lax
from jax.experimental import pallas as pl
from jax.experimental.pallas import tpu as pltpu
from jax.experimental.pallas import tpu_sc as plsc

EPS_LN = 1e-6

_SC_CORES = 2
_SC_SUBCORES = 16
_SC_LANES = 16
_NW = _SC_CORES * _SC_SUBCORES


def _knn_body(npow2_bits, ct_ref, xt_ref, i_ref, w_ref):
    n = ct_ref.shape[1]
    tm = xt_ref.shape[0]
    cx = ct_ref[0:1, :]
    cy = ct_ref[1:2, :]
    cz = ct_ref[2:3, :]
    tx = xt_ref[:, 0:1]
    ty = xt_ref[:, 1:2]
    tz = xt_ref[:, 2:3]
    dx = tx - cx
    dy = ty - cy
    dz = tz - cz
    d2 = dx * dx + dy * dy + dz * dz
    nb = 128
    bigf = jnp.float32(3e9)
    m1 = jnp.full((tm, nb), bigf, jnp.float32)
    m2 = m1
    m3 = m1
    flag = 1 << 30
    for g in range(n // nb):
        lane_c = (jax.lax.broadcasted_iota(jnp.int32, (tm, nb), 1)
                  + (g * nb | flag))
        kc_i = (d2[:, g * nb:(g + 1) * nb] << npow2_bits) | lane_c
        kc = jax.lax.bitcast_convert_type(kc_i, jnp.float32)
        lo = jnp.minimum(m1, kc)
        hi = jnp.maximum(m1, kc)
        m1 = lo
        lo = jnp.minimum(m2, hi)
        hi = jnp.maximum(m2, hi)
        m2 = lo
        m3 = jnp.minimum(m3, hi)
    ks = jnp.concatenate([m1, m2, m3], axis=1)
    k1 = jnp.min(ks, axis=1, keepdims=True)
    ks = jnp.where(ks == k1, bigf, ks)
    k2 = jnp.min(ks, axis=1, keepdims=True)
    ks = jnp.where(ks == k2, bigf, ks)
    k3 = jnp.min(ks, axis=1, keepdims=True)

    mask = jnp.int32((1 << npow2_bits) - 1)
    w_list = []
    idx_list = []
    for k in (k1, k2, k3):
        ki = jax.lax.bitcast_convert_type(k, jnp.int32) & (flag - 1)
        idx_list.append(ki & mask)
        d2f = (ki >> npow2_bits).astype(jnp.float32)
        w_list.append(1.0 / (d2f + 1e-8))
    norm = w_list[0] + w_list[1] + w_list[2]
    w_list = [w / norm for w in w_list]

    zi = jnp.zeros((tm, 5), jnp.int32)
    zf = jnp.zeros((tm, 5), jnp.float32)
    i_ref[...] = jnp.concatenate(idx_list + [zi], axis=1)
    w_ref[...] = jnp.concatenate(w_list + [zf], axis=1)


def _combine_body(g0_ref, g1_ref, g2_ref, w_ref, ga_ref, be_ref, o_ref):
    acc = (g0_ref[...] * w_ref[:, 0:1]
           + g1_ref[...] * w_ref[:, 1:2]
           + g2_ref[...] * w_ref[:, 2:3])
    mu = jnp.mean(acc, axis=1, keepdims=True)
    xc = acc - mu
    var = jnp.mean(xc * xc, axis=1, keepdims=True)
    y = xc / jnp.sqrt(var + EPS_LN) * ga_ref[0:1, :] + be_ref[0:1, :]
    o_ref[...] = y * 0.5 * (1.0 + jax.lax.erf(y * 0.7071067811865476))


def _make_sc_gather(n, m, c):
    rw = m // _NW
    g = min(128, rw)
    mesh = plsc.VectorSubcoreMesh(core_axis_name="c", subcore_axis_name="s")

    out_row = jax.ShapeDtypeStruct((m, c), jnp.float32)

    @functools.partial(
        pl.kernel,
        mesh=mesh,
        compiler_params=pltpu.CompilerParams(needs_layout_passes=False),
        out_type=(out_row, out_row, out_row),
        scratch_types=[
            pltpu.VMEM((n,), jnp.int32),
            pltpu.VMEM((rw,), jnp.int32),
            pltpu.VMEM((g,), jnp.int32),
            pltpu.VMEM((g, c), jnp.float32),
            pltpu.SemaphoreType.DMA,
        ],
    )
    def sc_gather(order_hbm, s0_hbm, s1_hbm, s2_hbm, feats_hbm,
                  o0_hbm, o1_hbm, o2_hbm,
                  order_v, sidx_v, gidx_v, rows_v, sem):
        wid = lax.axis_index("s") * _SC_CORES + lax.axis_index("c")
        base = wid * rw
        pltpu.sync_copy(order_hbm, order_v)
        for s_hbm, o_hbm in ((s0_hbm, o0_hbm), (s1_hbm, o1_hbm),
                             (s2_hbm, o2_hbm)):
            pltpu.sync_copy(s_hbm.at[pl.ds(base, rw)], sidx_v)
            for ch in range(rw // g):
                for q in range(g // _SC_LANES):
                    sl = pl.ds(q * _SC_LANES, _SC_LANES)
                    iv = sidx_v[pl.ds(ch * g + q * _SC_LANES, _SC_LANES)]
                    gidx_v[sl] = plsc.load_gather(order_v, [iv])
                pltpu.async_copy(feats_hbm.at[gidx_v], rows_v, sem).wait()
                pltpu.sync_copy(rows_v, o_hbm.at[pl.ds(base + ch * g, g)])

    return sc_gather


def _sort_keys(coord):
    step = coord.max() + 1
    return coord[..., 0] + coord[..., 1] * step + coord[..., 2] * step * step


def kernel(features, coords, xyz_t, gamma, beta):
    b, n, c = features.shape
    m = xyz_t.shape[1]
    r = b * m
    npow2_bits = max(1, (n - 1).bit_length())

    order = jnp.argsort(_sort_keys(coords), axis=1)
    order_t = jnp.argsort(_sort_keys(xyz_t), axis=1)
    c_sorted = jnp.take_along_axis(coords, order[..., None], axis=1)
    xt_s = jnp.take_along_axis(xyz_t, order_t[..., None], axis=1)

    ct_pad = jnp.zeros((b, 8, n), jnp.int32).at[:, 0:3, :].set(
        c_sorted.transpose(0, 2, 1))
    xt_pad = jnp.zeros((b, m, 8), jnp.int32).at[:, :, 0:3].set(xt_s)

    tm = 256
    knn_call = pl.pallas_call(
        functools.partial(_knn_body, npow2_bits),
        grid=(m // tm,),
        in_specs=[
            pl.BlockSpec((8, n), lambda j: (0, 0)),
            pl.BlockSpec((tm, 8), lambda j: (j, 0)),
        ],
        out_specs=[
            pl.BlockSpec((tm, 8), lambda j: (j, 0)),
            pl.BlockSpec((tm, 8), lambda j: (j, 0)),
        ],
        out_shape=[
            jax.ShapeDtypeStruct((m, 8), jnp.int32),
            jax.ShapeDtypeStruct((m, 8), jnp.float32),
        ],
    )
    sc_call = _make_sc_gather(n, m, c)
    tr = 1024
    combine_call = pl.pallas_call(
        _combine_body,
        grid=(m // tr,),
        in_specs=[
            pl.BlockSpec((tr, c), lambda i: (i, 0)),
            pl.BlockSpec((tr, c), lambda i: (i, 0)),
            pl.BlockSpec((tr, c), lambda i: (i, 0)),
            pl.BlockSpec((tr, 8), lambda i: (i, 0)),
            pl.BlockSpec((1, c), lambda i: (0, 0)),
            pl.BlockSpec((1, c), lambda i: (0, 0)),
        ],
        out_specs=pl.BlockSpec((tr, c), lambda i: (i, 0)),
        out_shape=jax.ShapeDtypeStruct((m, c), jnp.float32),
    )
    gamma2 = gamma.reshape(1, c)
    beta2 = beta.reshape(1, c)
    order32 = order.astype(jnp.int32)

    feats_out = []
    for bi in range(b):
        sidx8, w8 = knn_call(ct_pad[bi], xt_pad[bi])
        g0, g1, g2 = sc_call(
            order32[bi], sidx8[:, 0], sidx8[:, 1], sidx8[:, 2],
            features[bi])
        feats_out.append(combine_call(g0, g1, g2, w8, gamma2, beta2))
    out_feats = jnp.concatenate(feats_out, axis=0)

    xt_f = xt_s.reshape(r, 3).astype(jnp.float32)
    bcol = jnp.repeat(jnp.arange(b, dtype=jnp.float32), m)[:, None]
    out_coords = jnp.concatenate([bcol, xt_f], axis=1)
    return out_feats, out_coords


# speedup vs baseline: 5.9277x; 1.0260x over previous
import functools

import jax
import jax.numpy as jnp
from jax import lax
from jax.experimental import pallas as pl
from jax.experimental.pallas import tpu as pltpu
from jax.experimental.pallas import tpu_sc as plsc

EPS_LN = 1e-6

_SC_CORES = 2
_SC_SUBCORES = 16
_SC_LANES = 16
_NW = _SC_CORES * _SC_SUBCORES


def _knn_body(npow2_bits, ct_ref, xt_ref, i_ref, w_ref):
    n = ct_ref.shape[1]
    tm = xt_ref.shape[0]
    cx = ct_ref[0:1, :]
    cy = ct_ref[1:2, :]
    cz = ct_ref[2:3, :]
    tx = xt_ref[:, 0:1]
    ty = xt_ref[:, 1:2]
    tz = xt_ref[:, 2:3]
    dx = tx - cx
    dy = ty - cy
    dz = tz - cz
    d2 = dx * dx + dy * dy + dz * dz
    nb = 128
    bigf = jnp.float32(3e9)
    m1 = jnp.full((tm, nb), bigf, jnp.float32)
    m2 = m1
    m3 = m1
    flag = 1 << 30
    for g in range(n // nb):
        lane_c = (jax.lax.broadcasted_iota(jnp.int32, (tm, nb), 1)
                  + (g * nb | flag))
        kc_i = d2[:, g * nb:(g + 1) * nb] | lane_c
        kc = jax.lax.bitcast_convert_type(kc_i, jnp.float32)
        lo = jnp.minimum(m1, kc)
        hi = jnp.maximum(m1, kc)
        m1 = lo
        lo = jnp.minimum(m2, hi)
        hi = jnp.maximum(m2, hi)
        m2 = lo
        m3 = jnp.minimum(m3, hi)
    ks = jnp.concatenate([m1, m2, m3], axis=1)
    k1 = jnp.min(ks, axis=1, keepdims=True)
    ks = jnp.where(ks == k1, bigf, ks)
    k2 = jnp.min(ks, axis=1, keepdims=True)
    ks = jnp.where(ks == k2, bigf, ks)
    k3 = jnp.min(ks, axis=1, keepdims=True)

    mask = jnp.int32((1 << npow2_bits) - 1)
    w_list = []
    idx_list = []
    for k in (k1, k2, k3):
        ki = jax.lax.bitcast_convert_type(k, jnp.int32) & (flag - 1)
        idx_list.append(ki & mask)
        d2f = (ki >> npow2_bits).astype(jnp.float32)
        w_list.append(1.0 / (d2f + 1e-8))
    norm = w_list[0] + w_list[1] + w_list[2]
    w_list = [w / norm for w in w_list]

    zi = jnp.zeros((tm, 5), jnp.int32)
    zf = jnp.zeros((tm, 5), jnp.float32)
    i_ref[...] = jnp.concatenate(idx_list + [zi], axis=1)
    w_ref[...] = jnp.concatenate(w_list + [zf], axis=1)


def _combine_body(g0_ref, g1_ref, g2_ref, w_ref, ga_ref, be_ref, o_ref):
    acc = (g0_ref[...] * w_ref[:, 0:1]
           + g1_ref[...] * w_ref[:, 1:2]
           + g2_ref[...] * w_ref[:, 2:3])
    mu = jnp.mean(acc, axis=1, keepdims=True)
    xc = acc - mu
    var = jnp.mean(xc * xc, axis=1, keepdims=True)
    y = xc / jnp.sqrt(var + EPS_LN) * ga_ref[0:1, :] + be_ref[0:1, :]
    o_ref[...] = y * 0.5 * (1.0 + jax.lax.erf(y * 0.7071067811865476))


def _make_sc_gather(n, m, c):
    rw = m // _NW
    g = min(128, rw)
    mesh = plsc.VectorSubcoreMesh(core_axis_name="c", subcore_axis_name="s")

    out_row = jax.ShapeDtypeStruct((m, c), jnp.float32)

    @functools.partial(
        pl.kernel,
        mesh=mesh,
        compiler_params=pltpu.CompilerParams(needs_layout_passes=False),
        out_type=(out_row, out_row, out_row),
        scratch_types=[
            pltpu.VMEM((n,), jnp.int32),
            pltpu.VMEM((rw,), jnp.int32),
            pltpu.VMEM((g,), jnp.int32),
            pltpu.VMEM((g, c), jnp.float32),
            pltpu.SemaphoreType.DMA,
        ],
    )
    def sc_gather(order_hbm, s0_hbm, s1_hbm, s2_hbm, feats_hbm,
                  o0_hbm, o1_hbm, o2_hbm,
                  order_v, sidx_v, gidx_v, rows_v, sem):
        wid = lax.axis_index("s") * _SC_CORES + lax.axis_index("c")
        base = wid * rw
        pltpu.sync_copy(order_hbm, order_v)
        for s_hbm, o_hbm in ((s0_hbm, o0_hbm), (s1_hbm, o1_hbm),
                             (s2_hbm, o2_hbm)):
            pltpu.sync_copy(s_hbm.at[pl.ds(base, rw)], sidx_v)
            for ch in range(rw // g):
                for q in range(g // _SC_LANES):
                    sl = pl.ds(q * _SC_LANES, _SC_LANES)
                    iv = sidx_v[pl.ds(ch * g + q * _SC_LANES, _SC_LANES)]
                    gidx_v[sl] = plsc.load_gather(order_v, [iv])
                pltpu.async_copy(feats_hbm.at[gidx_v], rows_v, sem).wait()
                pltpu.sync_copy(rows_v, o_hbm.at[pl.ds(base + ch * g, g)])

    return sc_gather


def _sort_keys(coord):
    step = coord.max() + 1
    return coord[..., 0] + coord[..., 1] * step + coord[..., 2] * step * step


def kernel(features, coords, xyz_t, gamma, beta):
    b, n, c = features.shape
    m = xyz_t.shape[1]
    r = b * m
    npow2_bits = 2 * ((max(1, (n - 1).bit_length()) + 1) // 2)

    order = jnp.argsort(_sort_keys(coords), axis=1)
    order_t = jnp.argsort(_sort_keys(xyz_t), axis=1)
    c_sorted = jnp.take_along_axis(coords, order[..., None], axis=1)
    xt_s = jnp.take_along_axis(xyz_t, order_t[..., None], axis=1)

    half = npow2_bits // 2
    ct_pad = jnp.zeros((b, 8, n), jnp.int32).at[:, 0:3, :].set(
        c_sorted.transpose(0, 2, 1) << half)
    xt_pad = jnp.zeros((b, m, 8), jnp.int32).at[:, :, 0:3].set(xt_s << half)

    tm = 256
    knn_call = pl.pallas_call(
        functools.partial(_knn_body, npow2_bits),
        grid=(m // tm,),
        in_specs=[
            pl.BlockSpec((8, n), lambda j: (0, 0)),
            pl.BlockSpec((tm, 8), lambda j: (j, 0)),
        ],
        out_specs=[
            pl.BlockSpec((tm, 8), lambda j: (j, 0)),
            pl.BlockSpec((tm, 8), lambda j: (j, 0)),
        ],
        out_shape=[
            jax.ShapeDtypeStruct((m, 8), jnp.int32),
            jax.ShapeDtypeStruct((m, 8), jnp.float32),
        ],
    )
    sc_call = _make_sc_gather(n, m, c)
    tr = 1024
    combine_call = pl.pallas_call(
        _combine_body,
        grid=(m // tr,),
        in_specs=[
            pl.BlockSpec((tr, c), lambda i: (i, 0)),
            pl.BlockSpec((tr, c), lambda i: (i, 0)),
            pl.BlockSpec((tr, c), lambda i: (i, 0)),
            pl.BlockSpec((tr, 8), lambda i: (i, 0)),
            pl.BlockSpec((1, c), lambda i: (0, 0)),
            pl.BlockSpec((1, c), lambda i: (0, 0)),
        ],
        out_specs=pl.BlockSpec((tr, c), lambda i: (i, 0)),
        out_shape=jax.ShapeDtypeStruct((m, c), jnp.float32),
    )
    gamma2 = gamma.reshape(1, c)
    beta2 = beta.reshape(1, c)
    order32 = order.astype(jnp.int32)

    feats_out = []
    for bi in range(b):
        sidx8, w8 = knn_call(ct_pad[bi], xt_pad[bi])
        g0, g1, g2 = sc_call(
            order32[bi], sidx8[:, 0], sidx8[:, 1], sidx8[:, 2],
            features[bi])
        feats_out.append(combine_call(g0, g1, g2, w8, gamma2, beta2))
    out_feats = jnp.concatenate(feats_out, axis=0)

    xt_f = xt_s.reshape(r, 3).astype(jnp.float32)
    bcol = jnp.repeat(jnp.arange(b, dtype=jnp.float32), m)[:, None]
    out_coords = jnp.concatenate([bcol, xt_f], axis=1)
    return out_feats, out_coords
